# baseline (speedup 1.0000x reference)
import jax
import jax.numpy as jnp
from jax import lax
from jax.experimental import pallas as pl
from jax.experimental.pallas import tpu as pltpu
from jax.experimental.pallas import tpu_sc as plsc

_NUM_ACT = 96
_SIZE = 101
_HALF = _SIZE // 2
_INV_GRID = 12.5
_CWIN = 128

_N = 4 * _NUM_ACT * 224 * 224
_NTILES = 32
_SLAB = 224 * 224
_CH = _SLAB // 8
_CPS = _SLAB // _CH
_VREGS = _CH // 16
_NBUF = 4
_NSLAB = _N // _SLAB
_XROWS = _SLAB // 128
_SC_SLABS = 240
_TC_SLABS = _NSLAB - _SC_SLABS
_SC_CHUNKS = _SC_SLABS * _CPS // _NTILES


def _tec_body(x_hbm, cv_hbm, sc_hbm, zk_hbm, out_hbm,
              xb0, xb1, xb2, xb3, ob0, ob1, ob2, ob3,
              cv_raw, sc_v, zk_v, cvp, dp,
              is0, is1, is2, is3, os0, os1, os2, os3):
    _CHUNKS = _SC_CHUNKS
    info = plsc.get_sparse_core_info()
    nc = info.num_cores
    wid = lax.axis_index("s") * nc + lax.axis_index("c")

    xbufs = (xb0, xb1, xb2, xb3)
    obufs = (ob0, ob1, ob2, ob3)
    isems = (is0, is1, is2, is3)
    osems = (os0, os1, os2, os3)

    def in_copy(k, b):
        return pltpu.make_async_copy(
            x_hbm.at[wid * _CHUNKS + k], xbufs[b], isems[b])

    def out_copy(k, b):
        return pltpu.make_async_copy(
            obufs[b], out_hbm.at[wid * _CHUNKS + k], osems[b])

    for b in range(_NBUF):
        in_copy(b, b).start()

    pltpu.sync_copy(cv_hbm, cv_raw.at[pl.ds(0, _NUM_ACT * _SIZE)])
    pltpu.sync_copy(sc_hbm, sc_v.at[pl.ds(0, _NUM_ACT)])
    pltpu.sync_copy(zk_hbm, zk_v.at[pl.ds(0, _NUM_ACT)])

    lane = lax.iota(jnp.int32, 16)

    def prep(c, carry):
        ci = jnp.full((16,), c, jnp.int32)
        zkv = plsc.load_gather(zk_v, [ci])
        rsv = 1.0 / plsc.load_gather(sc_v, [ci])
        b0 = zkv - _HALF
        for j in range(_CWIN // 16):
            src = b0 + (j * 16) + lane
            v0 = plsc.load_gather(cv_raw, [src])
            v1 = plsc.load_gather(cv_raw, [src + 1])
            dst = c * _CWIN + j * 16
            cvp[pl.ds(dst, 16)] = v0 * rsv
            dp[pl.ds(dst, 16)] = (v1 - v0) * rsv
        return carry

    lax.fori_loop(0, _NUM_ACT, prep, 0)

    def compute(k, xbuf, obuf):
        c = ((wid * _CHUNKS + k) // _CPS) % _NUM_ACT
        ci = jnp.full((16,), c, jnp.int32)
        scv = plsc.load_gather(sc_v, [ci])
        a = scv * _INV_GRID
        cb = jnp.full((16,), c * _CWIN, jnp.int32)

        @plsc.parallel_loop(0, _VREGS, 1, unroll=8)
        def vec(i):
            xv = xbuf[pl.ds(i * 16, 16)]
            s = xv * a + float(_HALF)
            u = jnp.clip(s, 0.0, float(_SIZE - 2))
            iu = u.astype(jnp.int32)
            fu = iu.astype(jnp.float32)
            fr = s - fu
            ix = cb + iu
            g0 = plsc.load_gather(cvp, [ix])
            dl = plsc.load_gather(dp, [ix])
            obuf[pl.ds(i * 16, 16)] = g0 + fr * dl

    def ring(p, carry):
        k0 = p * _NBUF
        for b in range(_NBUF):
            k = k0 + b
            in_copy(k, b).wait()

            @pl.when(k >= _NBUF)
            def _wait_out():
                out_copy(k - _NBUF, b).wait()

            compute(k, xbufs[b], obufs[b])
            out_copy(k, b).start()

            @pl.when(k + _NBUF < _CHUNKS)
            def _next_in():
                in_copy(k + _NBUF, b).start()
        return carry

    lax.fori_loop(0, _CHUNKS // _NBUF, ring, 0)
    for b in range(_NBUF):
        out_copy(_CHUNKS - _NBUF + b, b).wait()


def _spline_sc(xflat, cv, scv, zk):
    run = pl.kernel(
        _tec_body,
        out_type=jax.ShapeDtypeStruct((_SC_SLABS * _CPS, _CH), jnp.float32),
        mesh=plsc.VectorSubcoreMesh(core_axis_name="c", subcore_axis_name="s"),
        compiler_params=pltpu.CompilerParams(needs_layout_passes=False),
        scratch_types=(
            [pltpu.VMEM((_CH,), jnp.float32)] * (2 * _NBUF)
            + [
                pltpu.VMEM((9728,), jnp.float32),
                pltpu.VMEM((128,), jnp.float32),
                pltpu.VMEM((128,), jnp.int32),
                pltpu.VMEM((_NUM_ACT * _CWIN,), jnp.float32),
                pltpu.VMEM((_NUM_ACT * _CWIN,), jnp.float32),
            ]
            + [pltpu.SemaphoreType.DMA] * (2 * _NBUF)
        ),
    )
    return run(xflat, cv, scv, zk)


_NSLAB = _N // _SLAB
_XROWS = _SLAB // 128


def _tc_tables(cv, scv, zk):
    j = jnp.arange(_CWIN, dtype=jnp.int32)
    idx = zk[:, None] - _HALF + j[None, :]
    idx = jnp.clip(idx, 0, cv.shape[0] - 2)
    v0 = cv[idx]
    sl = cv[idx + 1] - v0
    rs = (1.0 / scv)[:, None]
    return (v0 * rs)[:, None, :], (sl * rs)[:, None, :], scv * _INV_GRID


def _tc_body(a_ref, vals_ref, slope_ref, x_ref, o_ref):
    c = lax.rem(pl.program_id(0), _NUM_ACT)
    a = a_ref[c]
    vb = jnp.broadcast_to(vals_ref[0], (_XROWS, _CWIN))
    sb = jnp.broadcast_to(slope_ref[0], (_XROWS, _CWIN))
    xv = x_ref[0]
    s = xv * a + float(_HALF)
    u = jnp.clip(s, 0.0, float(_SIZE - 2))
    iu = u.astype(jnp.int32)
    fr = s - iu.astype(jnp.float32)
    g0 = jnp.take_along_axis(vb, iu, axis=1)
    sl = jnp.take_along_axis(sb, iu, axis=1)
    o_ref[0] = g0 + fr * sl


def _spline_tc(x3, cv, scv, zk):
    nslab = x3.shape[0]
    vals, slope, a = _tc_tables(cv, scv, zk)
    cmap = lambda i: (lax.rem(i + _SC_SLABS, _NUM_ACT), 0, 0)
    return pl.pallas_call(
        _tc_body,
        grid=(nslab,),
        in_specs=[
            pl.BlockSpec(memory_space=pltpu.SMEM),
            pl.BlockSpec((1, 1, _CWIN), cmap),
            pl.BlockSpec((1, 1, _CWIN), cmap),
            pl.BlockSpec((1, _XROWS, 128), lambda i: (i, 0, 0)),
        ],
        out_specs=pl.BlockSpec((1, _XROWS, 128), lambda i: (i, 0, 0)),
        out_shape=jax.ShapeDtypeStruct((nslab, _XROWS, 128), jnp.float32),
    )(a, vals, slope, x3)


@jax.jit
def _spline_hybrid(x2, cv, scv, zk):
    xsc = x2[:_SC_SLABS].reshape(_SC_SLABS * _CPS, _CH)
    xtc = x2[_SC_SLABS:].reshape(_TC_SLABS, _XROWS, 128)
    osc = _spline_sc(xsc, cv, scv, zk)
    otc = _spline_tc(xtc, cv, scv, zk)
    return jnp.concatenate(
        [osc.reshape(_SC_SLABS, _SLAB), otc.reshape(_TC_SLABS, _SLAB)], axis=0)


def kernel(x, coefficients_vect, scaling_coeffs_vect, zero_knot_indexes):
    out = _spline_hybrid(
        x.reshape(_NSLAB, _SLAB),
        coefficients_vect,
        scaling_coeffs_vect.reshape(-1),
        zero_knot_indexes.astype(jnp.int32),
    )
    return out.reshape(x.shape)


# speedup vs baseline: 1.7389x; 1.7389x over previous
import jax
import jax.numpy as jnp
from jax import lax
from jax.experimental import pallas as pl
from jax.experimental.pallas import tpu as pltpu
from jax.experimental.pallas import tpu_sc as plsc

_NUM_ACT = 96
_SIZE = 101
_HALF = _SIZE // 2
_INV_GRID = 12.5
_CWIN = 128

_N = 4 * _NUM_ACT * 224 * 224
_NTILES = 32
_SLAB = 224 * 224
_CH = _SLAB // 4
_CPS = _SLAB // _CH
_VREGS = _CH // 16
_NBUF = 3
_NSLAB = _N // _SLAB
_CHUNKS = _NSLAB * _CPS // _NTILES


def _tec_body(x_hbm, cv_hbm, sc_hbm, zk_hbm, out_hbm, *scr):
    info = plsc.get_sparse_core_info()
    nc = info.num_cores
    wid = lax.axis_index("s") * nc + lax.axis_index("c")

    xbufs = scr[0:_NBUF]
    obufs = scr[_NBUF:2 * _NBUF]
    cv_raw, sc_v, zk_v, cvp, dp = scr[2 * _NBUF:2 * _NBUF + 5]
    isems = scr[2 * _NBUF + 5:3 * _NBUF + 5]
    osems = scr[3 * _NBUF + 5:4 * _NBUF + 5]

    def in_copy(k, b):
        return pltpu.make_async_copy(
            x_hbm.at[wid * _CHUNKS + k], xbufs[b], isems[b])

    def out_copy(k, b):
        return pltpu.make_async_copy(
            obufs[b], out_hbm.at[wid * _CHUNKS + k], osems[b])

    for b in range(_NBUF):
        in_copy(b, b).start()

    pltpu.sync_copy(cv_hbm, cv_raw.at[pl.ds(0, _NUM_ACT * _SIZE)])
    pltpu.sync_copy(sc_hbm, sc_v.at[pl.ds(0, _NUM_ACT)])
    pltpu.sync_copy(zk_hbm, zk_v.at[pl.ds(0, _NUM_ACT)])

    lane = lax.iota(jnp.int32, 16)

    def prep(c, carry):
        ci = jnp.full((16,), c, jnp.int32)
        zkv = plsc.load_gather(zk_v, [ci])
        rsv = 1.0 / plsc.load_gather(sc_v, [ci])
        b0 = zkv - _HALF
        for j in range(_CWIN // 16):
            src = b0 + (j * 16) + lane
            v0 = plsc.load_gather(cv_raw, [src])
            v1 = plsc.load_gather(cv_raw, [src + 1])
            dst = c * _CWIN + j * 16
            cvp[pl.ds(dst, 16)] = v0 * rsv
            dp[pl.ds(dst, 16)] = (v1 - v0) * rsv
        return carry

    lax.fori_loop(0, _NUM_ACT, prep, 0)

    def compute(k, xbuf, obuf):
        c = ((wid * _CHUNKS + k) // _CPS) % _NUM_ACT
        ci = jnp.full((16,), c, jnp.int32)
        scv = plsc.load_gather(sc_v, [ci])
        a = scv * _INV_GRID
        cb = jnp.full((16,), c * _CWIN, jnp.int32)

        @plsc.parallel_loop(0, _VREGS, 1, unroll=8)
        def vec(i):
            xv = xbuf[pl.ds(i * 16, 16)]
            s = xv * a + float(_HALF)
            u = jnp.clip(s, 0.0, float(_SIZE - 2))
            iu = u.astype(jnp.int32)
            fu = iu.astype(jnp.float32)
            fr = s - fu
            ix = cb + iu
            g0 = plsc.load_gather(cvp, [ix])
            dl = plsc.load_gather(dp, [ix])
            obuf[pl.ds(i * 16, 16)] = g0 + fr * dl

    def ring(p, carry):
        k0 = p * _NBUF
        for b in range(_NBUF):
            k = k0 + b
            in_copy(k, b).wait()

            @pl.when(k >= _NBUF)
            def _wait_out():
                out_copy(k - _NBUF, b).wait()

            compute(k, xbufs[b], obufs[b])
            out_copy(k, b).start()

            @pl.when(k + _NBUF < _CHUNKS)
            def _next_in():
                in_copy(k + _NBUF, b).start()
        return carry

    lax.fori_loop(0, _CHUNKS // _NBUF, ring, 0)
    for b in range(_NBUF):
        out_copy(_CHUNKS - _NBUF + b, b).wait()


@jax.jit
def _spline_sc(xflat, cv, scv, zk):
    run = pl.kernel(
        _tec_body,
        out_type=jax.ShapeDtypeStruct((_NSLAB * _CPS, _CH), jnp.float32),
        mesh=plsc.VectorSubcoreMesh(core_axis_name="c", subcore_axis_name="s"),
        compiler_params=pltpu.CompilerParams(needs_layout_passes=False),
        scratch_types=(
            [pltpu.VMEM((_CH,), jnp.float32)] * (2 * _NBUF)
            + [
                pltpu.VMEM((9728,), jnp.float32),
                pltpu.VMEM((128,), jnp.float32),
                pltpu.VMEM((128,), jnp.int32),
                pltpu.VMEM((_NUM_ACT * _CWIN,), jnp.float32),
                pltpu.VMEM((_NUM_ACT * _CWIN,), jnp.float32),
            ]
            + [pltpu.SemaphoreType.DMA] * (2 * _NBUF)
        ),
    )
    return run(xflat, cv, scv, zk)


def kernel(x, coefficients_vect, scaling_coeffs_vect, zero_knot_indexes):
    out = _spline_sc(
        x.reshape(_NSLAB * _CPS, _CH),
        coefficients_vect,
        scaling_coeffs_vect.reshape(-1),
        zero_knot_indexes.astype(jnp.int32),
    )
    return out.reshape(x.shape)
